# Initial kernel scaffold; baseline (speedup 1.0000x reference)
import functools

import jax
import jax.numpy as jnp
from jax import lax
from jax.experimental import pallas as pl
from jax.experimental.pallas import tpu as pltpu
from jax.experimental.pallas import tpu_sc as plsc

_NU = 4000
_NI = 6000
_N = _NU + _NI
_D = 256
_E = 160000
_NW = 32
_K = 64


def _make_layer():
    mesh = plsc.VectorSubcoreMesh(core_axis_name="c", subcore_axis_name="s")

    @functools.partial(
        pl.kernel,
        out_type=jax.ShapeDtypeStruct((_N * _D,), jnp.float32),
        mesh=mesh,
        scratch_types=[
            pltpu.VMEM((8,), jnp.int32),
            pltpu.VMEM((_K,), jnp.int32),
            pltpu.VMEM((_K,), jnp.int32),
            pltpu.VMEM((_K,), jnp.float32),
            pltpu.VMEM((_K, _D), jnp.float32),
            pltpu.VMEM((_D,), jnp.float32),
            pltpu.VMEM((8 * _D,), jnp.float32),
            pltpu.SemaphoreType.DMA,
        ],
    )
    def layer(bounds_hbm, src_hbm, dst_hbm, w_hbm, emb_hbm, out_hbm,
              bnds, srcb, dstb, wb, rows, acc, zblk, sem):
        cid = lax.axis_index("c")
        sid = lax.axis_index("s")
        wid = sid * 2 + cid

        pltpu.sync_copy(bounds_hbm.at[wid], bnds)
        e_lo = bnds[0]
        e_hi = bnds[1]
        n_lo = bnds[2]
        n_hi = bnds[3]

        zeros16 = jnp.zeros((16,), jnp.float32)
        for j in range(8 * _D // 16):
            zblk[pl.ds(16 * j, 16)] = zeros16
        for j in range(_D // 16):
            acc[pl.ds(16 * j, 16)] = zeros16

        cnt = n_hi - n_lo
        nfull = cnt // 8

        def zfull(c, carry):
            pltpu.sync_copy(zblk, out_hbm.at[pl.ds((n_lo + 8 * c) * _D, 8 * _D)])
            return carry

        lax.fori_loop(0, nfull, zfull, 0)

        def ztail(r, carry):
            pltpu.sync_copy(zblk.at[pl.ds(0, _D)],
                            out_hbm.at[pl.ds((n_lo + 8 * nfull + r) * _D, _D)])
            return carry

        lax.fori_loop(0, cnt - 8 * nfull, ztail, 0)

        base0 = (e_lo // _K) * _K
        nblk = jnp.where(e_lo < e_hi, (e_hi - base0 + _K - 1) // _K, 0)

        def blk(b, cur_dst):
            eb = base0 + b * _K
            c1 = pltpu.async_copy(src_hbm.at[pl.ds(eb, _K)], srcb, sem)
            c2 = pltpu.async_copy(dst_hbm.at[pl.ds(eb, _K)], dstb, sem)
            c3 = pltpu.async_copy(w_hbm.at[pl.ds(eb, _K)], wb, sem)
            c1.wait()
            c2.wait()
            c3.wait()
            pltpu.async_copy(emb_hbm.at[srcb], rows, sem).wait()

            def edge(i, cd):
                g = eb + i
                valid = jnp.logical_and(g >= e_lo, g < e_hi)

                def process(cd):
                    d = dstb[i]
                    w = wb[i]
                    changed = jnp.logical_and(d != cd, cd >= 0)

                    @pl.when(changed)
                    def _():
                        pltpu.sync_copy(acc, out_hbm.at[pl.ds(cd * _D, _D)])
                        for j in range(_D // 16):
                            acc[pl.ds(16 * j, 16)] = zeros16

                    for j in range(_D // 16):
                        v = rows[i, pl.ds(16 * j, 16)] * w
                        plsc.addupdate(acc.at[pl.ds(16 * j, 16)], v)
                    return d

                return lax.cond(valid, process, lambda cd: cd, cd)

            return lax.fori_loop(0, _K, edge, cur_dst)

        cur_dst = lax.fori_loop(0, nblk, blk, jnp.int32(-1))

        @pl.when(cur_dst >= 0)
        def _():
            pltpu.sync_copy(acc, out_hbm.at[pl.ds(cur_dst * _D, _D)])

    return layer


_layer = _make_layer()


def _norm_body(x_ref, o_ref):
    x = x_ref[...]
    s = jnp.sum(x * x, axis=1, keepdims=True)
    o_ref[...] = x / jnp.maximum(jnp.sqrt(s), 1e-12)


def _l2n(x, blk):
    m = x.shape[0]
    return pl.pallas_call(
        _norm_body,
        out_shape=jax.ShapeDtypeStruct(x.shape, x.dtype),
        grid=(m // blk,),
        in_specs=[pl.BlockSpec((blk, _D), lambda i: (i, 0))],
        out_specs=pl.BlockSpec((blk, _D), lambda i: (i, 0)),
    )(x)


def _final_body(a_ref, b_ref, c_ref, d_ref, o_ref):
    x = (a_ref[...] + b_ref[...] + c_ref[...] + d_ref[...]) * 0.25
    s = jnp.sum(x * x, axis=1, keepdims=True)
    o_ref[...] = x / jnp.maximum(jnp.sqrt(s), 1e-12)


def _finalize(a, b, c, d, blk=2000):
    spec = pl.BlockSpec((blk, _D), lambda i: (i, 0))
    return pl.pallas_call(
        _final_body,
        out_shape=jax.ShapeDtypeStruct((_N, _D), jnp.float32),
        grid=(_N // blk,),
        in_specs=[spec, spec, spec, spec],
        out_specs=spec,
    )(a, b, c, d)


def kernel(edge_index, edge_weight, user_emb_w, item_emb_w):
    src = edge_index[0].astype(jnp.int32)
    dst = edge_index[1].astype(jnp.int32)

    order = jnp.argsort(dst)
    srcs = src[order]
    dsts = dst[order]
    ws = edge_weight[order]
    srcp = jnp.concatenate([srcs, jnp.zeros((_K,), jnp.int32)])
    dstp = jnp.concatenate([dsts, jnp.zeros((_K,), jnp.int32)])
    wp = jnp.concatenate([ws, jnp.zeros((_K,), jnp.float32)])

    starts = jnp.arange(_NW, dtype=jnp.int32) * (_E // _NW)
    nlo = jnp.where(jnp.arange(_NW) == 0, 0, dsts[starts]).astype(jnp.int32)
    nhi = jnp.concatenate([nlo[1:], jnp.array([_N], jnp.int32)])
    elo = jnp.searchsorted(dsts, nlo, side="left").astype(jnp.int32)
    ehi = jnp.concatenate([elo[1:], jnp.array([_E], jnp.int32)])
    zeros = jnp.zeros((_NW,), jnp.int32)
    bounds = jnp.stack([elo, ehi, nlo, nhi, zeros, zeros, zeros, zeros], axis=1)

    emb0 = jnp.concatenate([_l2n(user_emb_w, 2000), _l2n(item_emb_w, 2000)],
                           axis=0)
    embs = [emb0]
    e = emb0
    for _ in range(3):
        e = _layer(bounds, srcp, dstp, wp, e).reshape(_N, _D)
        embs.append(e)
    final = _finalize(*embs)
    return final[:_NU], final[_NU:]


# speedup vs baseline: 1.5067x; 1.5067x over previous
import functools

import jax
import jax.numpy as jnp
from jax import lax
from jax.experimental import pallas as pl
from jax.experimental.pallas import tpu as pltpu
from jax.experimental.pallas import tpu_sc as plsc

_NU = 4000
_NI = 6000
_N = _NU + _NI
_D = 256
_E = 160000
_NW = 32
_K = 64


def _make_layer():
    mesh = plsc.VectorSubcoreMesh(core_axis_name="c", subcore_axis_name="s")

    @functools.partial(
        pl.kernel,
        out_type=jax.ShapeDtypeStruct((_N * _D,), jnp.float32),
        mesh=mesh,
        scratch_types=[
            pltpu.VMEM((16,), jnp.int32),
            pltpu.VMEM((_K,), jnp.int32),
            pltpu.VMEM((_K,), jnp.int32),
            pltpu.VMEM((_K,), jnp.float32),
            pltpu.VMEM((_K, _D), jnp.float32),
            pltpu.VMEM((_D,), jnp.float32),
            pltpu.VMEM((8 * _D,), jnp.float32),
            pltpu.SemaphoreType.DMA,
        ],
    )
    def layer(bounds_hbm, src_hbm, dst_hbm, w_hbm, emb_hbm, out_hbm,
              bnds, srcb, dstb, wb, rows, acc, zblk, sem):
        cid = lax.axis_index("c")
        sid = lax.axis_index("s")
        wid = sid * 2 + cid

        pltpu.sync_copy(bounds_hbm.at[wid], bnds)
        bv = bnds[...]
        e_lo = bv[0]
        e_hi = bv[1]
        n_lo = bv[2]
        n_hi = bv[3]

        zeros16 = jnp.zeros((16,), jnp.float32)
        for j in range(8 * _D // 16):
            zblk[pl.ds(16 * j, 16)] = zeros16
        for j in range(_D // 16):
            acc[pl.ds(16 * j, 16)] = zeros16

        cnt = n_hi - n_lo
        nfull = cnt // 8

        def zfull(c, carry):
            pltpu.sync_copy(zblk, out_hbm.at[pl.ds((n_lo + 8 * c) * _D, 8 * _D)])
            return carry

        lax.fori_loop(0, nfull, zfull, 0)

        def ztail(r, carry):
            pltpu.sync_copy(zblk.at[pl.ds(0, _D)],
                            out_hbm.at[pl.ds((n_lo + 8 * nfull + r) * _D, _D)])
            return carry

        lax.fori_loop(0, cnt - 8 * nfull, ztail, 0)

        base0 = (e_lo // _K) * _K
        nblk = jnp.where(e_lo < e_hi, (e_hi - base0 + _K - 1) // _K, 0)

        def blk(b, cur_dst):
            eb = base0 + b * _K
            c1 = pltpu.async_copy(src_hbm.at[pl.ds(eb, _K)], srcb, sem)
            c2 = pltpu.async_copy(dst_hbm.at[pl.ds(eb, _K)], dstb, sem)
            c3 = pltpu.async_copy(w_hbm.at[pl.ds(eb, _K)], wb, sem)
            c1.wait()
            c2.wait()
            c3.wait()
            pltpu.async_copy(emb_hbm.at[srcb], rows, sem).wait()

            def grp(q, cd):
                dvec = dstb[pl.ds(16 * q, 16)]
                wvec = wb[pl.ds(16 * q, 16)]
                for l in range(16):
                    i = 16 * q + l
                    g = eb + i
                    valid = jnp.logical_and(g >= e_lo, g < e_hi)
                    d = dvec[l]
                    w = wvec[l]

                    def process(cd, i=i, d=d, w=w):
                        changed = jnp.logical_and(d != cd, cd >= 0)

                        @pl.when(changed)
                        def _():
                            pltpu.sync_copy(acc, out_hbm.at[pl.ds(cd * _D, _D)])
                            for j in range(_D // 16):
                                acc[pl.ds(16 * j, 16)] = zeros16

                        for j in range(_D // 16):
                            v = rows[i, pl.ds(16 * j, 16)] * w
                            plsc.addupdate(acc.at[pl.ds(16 * j, 16)], v)
                        return d

                    cd = lax.cond(valid, process, lambda cd: cd, cd)
                return cd

            return lax.fori_loop(0, _K // 16, grp, cur_dst)

        cur_dst = lax.fori_loop(0, nblk, blk, jnp.int32(-1))

        @pl.when(cur_dst >= 0)
        def _():
            pltpu.sync_copy(acc, out_hbm.at[pl.ds(cur_dst * _D, _D)])

    return layer


_layer = _make_layer()


def _norm_body(x_ref, o_ref):
    x = x_ref[...]
    s = jnp.sum(x * x, axis=1, keepdims=True)
    o_ref[...] = x / jnp.maximum(jnp.sqrt(s), 1e-12)


def _l2n(x, blk):
    m = x.shape[0]
    return pl.pallas_call(
        _norm_body,
        out_shape=jax.ShapeDtypeStruct(x.shape, x.dtype),
        grid=(m // blk,),
        in_specs=[pl.BlockSpec((blk, _D), lambda i: (i, 0))],
        out_specs=pl.BlockSpec((blk, _D), lambda i: (i, 0)),
    )(x)


def _final_body(a_ref, b_ref, c_ref, d_ref, o_ref):
    x = (a_ref[...] + b_ref[...] + c_ref[...] + d_ref[...]) * 0.25
    s = jnp.sum(x * x, axis=1, keepdims=True)
    o_ref[...] = x / jnp.maximum(jnp.sqrt(s), 1e-12)


def _finalize(a, b, c, d, blk=2000):
    spec = pl.BlockSpec((blk, _D), lambda i: (i, 0))
    return pl.pallas_call(
        _final_body,
        out_shape=jax.ShapeDtypeStruct((_N, _D), jnp.float32),
        grid=(_N // blk,),
        in_specs=[spec, spec, spec, spec],
        out_specs=spec,
    )(a, b, c, d)


def kernel(edge_index, edge_weight, user_emb_w, item_emb_w):
    src = edge_index[0].astype(jnp.int32)
    dst = edge_index[1].astype(jnp.int32)

    order = jnp.argsort(dst)
    srcs = src[order]
    dsts = dst[order]
    ws = edge_weight[order]
    srcp = jnp.concatenate([srcs, jnp.zeros((_K,), jnp.int32)])
    dstp = jnp.concatenate([dsts, jnp.zeros((_K,), jnp.int32)])
    wp = jnp.concatenate([ws, jnp.zeros((_K,), jnp.float32)])

    starts = jnp.arange(_NW, dtype=jnp.int32) * (_E // _NW)
    nlo = jnp.where(jnp.arange(_NW) == 0, 0, dsts[starts]).astype(jnp.int32)
    nhi = jnp.concatenate([nlo[1:], jnp.array([_N], jnp.int32)])
    elo = jnp.searchsorted(dsts, nlo, side="left").astype(jnp.int32)
    ehi = jnp.concatenate([elo[1:], jnp.array([_E], jnp.int32)])
    zeros = jnp.zeros((_NW,), jnp.int32)
    bounds = jnp.stack([elo, ehi, nlo, nhi] + [zeros] * 12, axis=1)

    emb0 = jnp.concatenate([_l2n(user_emb_w, 2000), _l2n(item_emb_w, 2000)],
                           axis=0)
    embs = [emb0]
    e = emb0
    for _ in range(3):
        e = _layer(bounds, srcp, dstp, wp, e).reshape(_N, _D)
        embs.append(e)
    final = _finalize(*embs)
    return final[:_NU], final[_NU:]
